# baseline (speedup 1.0000x reference)
import functools

import jax
import jax.numpy as jnp
from jax import lax
from jax.experimental import pallas as pl
from jax.experimental.pallas import tpu as pltpu
from jax.experimental.pallas import tpu_sc as plsc

N_NODES = 10000
N_EDGES = 320000
NC = 2
NS = 16
N_PAD = 10240
ROWS_PER_TILE_PAD = N_PAD // NS
CK = 100
DCH = N_EDGES // NS // CK
CKP = 128
ECH = 160


@functools.lru_cache(maxsize=None)
def _mesh():
    return plsc.VectorSubcoreMesh(core_axis_name="c", subcore_axis_name="s",
                                  num_cores=NC, num_subcores=NS)


def _deg_body(src_hbm, dst_hbm, ns_hbm, nd_hbm, acc, idxm, ones_v, degv, zv,
              sm0, sm1, sm2, sm3):
    c = lax.axis_index("c")
    s = lax.axis_index("s")
    one = jnp.ones((16,), jnp.float32)
    zero = jnp.zeros((16,), jnp.float32)
    for j in range(112 // 16):
        ones_v[pl.ds(j * 16, 16)] = one
    for j in range(ROWS_PER_TILE_PAD // 16):
        zv[pl.ds(j * 16, 16)] = zero
    base = s * ROWS_PER_TILE_PAD
    pltpu.sync_copy(zv, acc.at[pl.ds(base, ROWS_PER_TILE_PAD)])

    @pl.when(c == 0)
    def _():
        pltpu.sync_copy(src_hbm.at[s], idxm)

    @pl.when(c != 0)
    def _():
        pltpu.sync_copy(dst_hbm.at[s], idxm)

    plsc.subcore_barrier()

    ones_c = ones_v.at[pl.ds(0, CK)]
    sems = (sm0, sm1, sm2, sm3)
    for b in range(4):
        pltpu.async_copy(ones_c, acc.at[idxm.at[b]], sems[b], add=True)

    def ring(k, carry):
        i = 4 * k
        for b in range(4):
            pltpu.make_async_copy(ones_c, acc.at[idxm.at[i + b]],
                                  sems[b]).wait()

            @pl.when(k + 1 < DCH // 4)
            def _():
                pltpu.async_copy(ones_c, acc.at[idxm.at[i + 4 + b]], sems[b],
                                 add=True)

        return carry

    lax.fori_loop(0, DCH // 4, ring, 0)
    plsc.subcore_barrier()

    pltpu.sync_copy(acc.at[pl.ds(base, ROWS_PER_TILE_PAD)], degv)

    def nbody(r, carry):
        dv = degv[pl.ds(r * 16, 16)]
        d = jnp.maximum(dv, 1.0)
        i32 = lax.bitcast_convert_type(d, jnp.int32)
        y = lax.bitcast_convert_type(jnp.int32(0x5F3759DF) - (i32 >> 1),
                                     jnp.float32)
        for _ in range(3):
            y = y * (1.5 - 0.5 * d * y * y)
        degv[pl.ds(r * 16, 16)] = jnp.where(dv > 0.0, y, 0.0)
        return carry

    lax.fori_loop(0, ROWS_PER_TILE_PAD // 16, nbody, 0)

    @pl.when(c == 0)
    def _():
        pltpu.sync_copy(degv, ns_hbm.at[pl.ds(base, ROWS_PER_TILE_PAD)])

    @pl.when(c != 0)
    def _():
        pltpu.sync_copy(degv, nd_hbm.at[pl.ds(base, ROWS_PER_TILE_PAD)])


@functools.lru_cache(maxsize=None)
def _deg_call():
    return pl.kernel(
        _deg_body,
        out_type=(jax.ShapeDtypeStruct((N_PAD,), jnp.float32),
                  jax.ShapeDtypeStruct((N_PAD,), jnp.float32)),
        mesh=_mesh(),
        scratch_types=[
            pltpu.VMEM_SHARED((N_PAD,), jnp.float32),
            pltpu.VMEM((DCH, CK), jnp.int32),
            pltpu.VMEM((112,), jnp.float32),
            pltpu.VMEM((ROWS_PER_TILE_PAD,), jnp.float32),
            pltpu.VMEM((ROWS_PER_TILE_PAD,), jnp.float32),
            pltpu.SemaphoreType.DMA,
            pltpu.SemaphoreType.DMA,
            pltpu.SemaphoreType.DMA,
            pltpu.SemaphoreType.DMA,
        ],
    )


def _gsh_body(FH, idx_halves, final, yh_hbm, src_hbm, dst_hbm, ns_hbm,
              nd_hbm, b_hbm, out_hbm, acc, ytab, idx_s, idx_d, rows0, rows1,
              ndv, bv, sg0, sg1):
    c = lax.axis_index("c")
    s = lax.axis_index("s")
    zero = jnp.zeros((16,), jnp.float32)

    def zb(r, carry):
        for j in range(FH // 16):
            rows0[r, pl.ds(j * 16, 16)] = zero
        return carry

    lax.fori_loop(0, CKP, zb, 0)
    base_rows = s * ROWS_PER_TILE_PAD
    for j in range(ROWS_PER_TILE_PAD // CKP):
        pltpu.sync_copy(rows0, acc.at[pl.ds(base_rows + j * CKP, CKP)])
    for blk in range(ROWS_PER_TILE_PAD // CKP):
        rb = base_rows + blk * CKP
        pltpu.sync_copy(yh_hbm.at[c, pl.ds(rb, CKP)], rows1)
        pltpu.sync_copy(ns_hbm.at[pl.ds(rb, CKP)], ndv)

        def scalep(r16, carry):
            nsr16 = ndv[pl.ds(r16 * 16, 16)]
            for rr in range(16):
                r = r16 * 16 + rr
                for j in range(FH // 16):
                    rows1[r, pl.ds(j * 16, 16)] = (
                        rows1[r, pl.ds(j * 16, 16)] * nsr16[rr])
            return carry

        lax.fori_loop(0, CKP // 16, scalep, 0)
        pltpu.sync_copy(rows1, ytab.at[pl.ds(rb, CKP)])
    nh = ECH // idx_halves

    def run_half(h):
        pltpu.sync_copy(src_hbm.at[s, pl.ds(h * nh, nh)], idx_s)
        pltpu.sync_copy(dst_hbm.at[s, pl.ds(h * nh, nh)], idx_d)
        if h == 0:
            plsc.subcore_barrier()
        pltpu.async_copy(ytab.at[idx_s.at[0]], rows0, sg0)

        def pair(k, carry):
            i = 2 * k
            pltpu.async_copy(ytab.at[idx_s.at[i + 1]], rows1, sg1)
            pltpu.make_async_copy(ytab.at[idx_s.at[i]], rows0, sg0).wait()
            pltpu.sync_copy(rows0, acc.at[idx_d.at[i]], add=True)

            @pl.when(k + 1 < nh // 2)
            def _():
                pltpu.async_copy(ytab.at[idx_s.at[i + 2]], rows0, sg0)

            pltpu.make_async_copy(ytab.at[idx_s.at[i + 1]], rows1, sg1).wait()
            pltpu.sync_copy(rows1, acc.at[idx_d.at[i + 1]], add=True)
            return carry

        lax.fori_loop(0, nh // 2, pair, 0)

    for h in range(idx_halves):
        run_half(h)
    plsc.subcore_barrier()

    pltpu.sync_copy(b_hbm.at[pl.ds(c * FH, FH)], bv)
    for blk in range(ROWS_PER_TILE_PAD // CKP):
        rb = base_rows + blk * CKP
        pltpu.sync_copy(acc.at[pl.ds(rb, CKP)], rows0)
        pltpu.sync_copy(nd_hbm.at[pl.ds(rb, CKP)], ndv)

        def rowp(r16, carry):
            ndr16 = ndv[pl.ds(r16 * 16, 16)]
            for rr in range(16):
                r = r16 * 16 + rr
                nd_s = ndr16[rr]
                for j in range(FH // 16):
                    v = rows0[r, pl.ds(j * 16, 16)]
                    rows0[r, pl.ds(j * 16, 16)] = jnp.maximum(
                        v * nd_s + bv[pl.ds(j * 16, 16)], 0.0)
            return carry

        lax.fori_loop(0, CKP // 16, rowp, 0)
        if not final:
            pltpu.sync_copy(rows0, out_hbm.at[c, pl.ds(rb, CKP)])
        else:
            rem = N_NODES % CKP

            @pl.when(rb + CKP <= N_NODES)
            def _():
                pltpu.sync_copy(rows0,
                                out_hbm.at[pl.ds(rb, CKP), pl.ds(c * FH, FH)])

            @pl.when(jnp.logical_and(rb < N_NODES, rb + CKP > N_NODES))
            def _():
                pltpu.sync_copy(rows0.at[pl.ds(0, rem)],
                                out_hbm.at[pl.ds(rb, rem), pl.ds(c * FH, FH)])


@functools.lru_cache(maxsize=None)
def _make_gsh(FH, idx_halves, final=False):
    if final:
        out_type = jax.ShapeDtypeStruct((N_NODES, 2 * FH), jnp.float32)
    else:
        out_type = jax.ShapeDtypeStruct((NC, N_PAD, FH), jnp.float32)
    return pl.kernel(
        functools.partial(_gsh_body, FH, idx_halves, final),
        out_type=out_type,
        mesh=_mesh(),
        scratch_types=[
            pltpu.VMEM_SHARED((N_PAD, FH), jnp.float32),
            pltpu.VMEM_SHARED((N_PAD, FH), jnp.float32),
            pltpu.VMEM((ECH // idx_halves, CKP), jnp.int32),
            pltpu.VMEM((ECH // idx_halves, CKP), jnp.int32),
            pltpu.VMEM((CKP, FH), jnp.float32),
            pltpu.VMEM((CKP, FH), jnp.float32),
            pltpu.VMEM((CKP,), jnp.float32),
            pltpu.VMEM((FH,), jnp.float32),
            pltpu.SemaphoreType.DMA,
            pltpu.SemaphoreType.DMA,
        ],
        compiler_params=pltpu.CompilerParams(use_tc_tiling_on_sc=False),
    )


def _prep_idx(src, dst):
    e_tile = N_EDGES // NS
    padt = ECH * CKP - e_tile
    zpad = jnp.zeros((NS, padt), jnp.int32)
    srcp = jnp.concatenate([src.reshape(NS, e_tile), zpad],
                           axis=1).reshape(NS, ECH, CKP)
    dpad = jnp.full((NS, padt), N_NODES, jnp.int32)
    dstp = jnp.concatenate([dst.reshape(NS, e_tile), dpad],
                           axis=1).reshape(NS, ECH, CKP)
    return srcp, dstp


def _tc1_body(x_ref, w_ref, o_ref):
    z = jnp.dot(x_ref[...], w_ref[...], preferred_element_type=jnp.float32)
    fh = z.shape[1] // 2
    o_ref[0, :N_NODES] = z[:, :fh]
    o_ref[1, :N_NODES] = z[:, fh:]


def _tc2_body(p_ref, w2_ref, o_ref):
    h = jnp.concatenate([p_ref[0, :N_NODES], p_ref[1, :N_NODES]], axis=1)
    z = jnp.dot(h, w2_ref[...], preferred_element_type=jnp.float32)
    fh = z.shape[1] // 2
    o_ref[0, :N_NODES] = z[:, :fh]
    o_ref[1, :N_NODES] = z[:, fh:]


def _tc1_call(x, w1):
    return pl.pallas_call(
        _tc1_body,
        out_shape=jax.ShapeDtypeStruct((2, N_PAD, w1.shape[1] // 2),
                                       jnp.float32),
    )(x, w1)


def _tc2_call(p, w2):
    return pl.pallas_call(
        _tc2_body,
        out_shape=jax.ShapeDtypeStruct((2, N_PAD, w2.shape[1] // 2),
                                       jnp.float32),
    )(p, w2)


def kernel(in_feat, edge_index, W1, b1, W2, b2):
    ei = edge_index.astype(jnp.int32)
    src = ei[0]
    dst = ei[1]
    srcd = src.reshape(NS, DCH, CK)
    dstd = dst.reshape(NS, DCH, CK)
    srcx, dstx = _prep_idx(src, dst)
    ns_pad, nd_pad = _deg_call()(srcd, dstd)
    z1h = _tc1_call(in_feat, W1)
    p1 = _make_gsh(64, 2)(z1h, srcx, dstx, ns_pad, nd_pad, b1)
    z2h = _tc2_call(p1, W2)
    return _make_gsh(32, 1, final=True)(z2h, srcx, dstx, ns_pad, nd_pad, b2)


# speedup vs baseline: 1.1627x; 1.1627x over previous
import functools

import jax
import jax.numpy as jnp
from jax import lax
from jax.experimental import pallas as pl
from jax.experimental.pallas import tpu as pltpu
from jax.experimental.pallas import tpu_sc as plsc

N_NODES = 10000
N_EDGES = 320000
NC = 2
NS = 16
N_PAD = 10240
ROWS_PER_TILE_PAD = N_PAD // NS
CK = 100
DCH = N_EDGES // NS // CK
CKP = 128
ECH = 160


@functools.lru_cache(maxsize=None)
def _mesh():
    return plsc.VectorSubcoreMesh(core_axis_name="c", subcore_axis_name="s",
                                  num_cores=NC, num_subcores=NS)


def _deg_body(src_hbm, dst_hbm, ns_hbm, nd_hbm, acc, idxm, ones_v, degv, zv,
              sm0, sm1, sm2, sm3):
    c = lax.axis_index("c")
    s = lax.axis_index("s")
    one = jnp.ones((16,), jnp.float32)
    zero = jnp.zeros((16,), jnp.float32)
    for j in range(112 // 16):
        ones_v[pl.ds(j * 16, 16)] = one
    for j in range(ROWS_PER_TILE_PAD // 16):
        zv[pl.ds(j * 16, 16)] = zero
    base = s * ROWS_PER_TILE_PAD
    pltpu.sync_copy(zv, acc.at[pl.ds(base, ROWS_PER_TILE_PAD)])

    @pl.when(c == 0)
    def _():
        pltpu.sync_copy(src_hbm.at[s], idxm)

    @pl.when(c != 0)
    def _():
        pltpu.sync_copy(dst_hbm.at[s], idxm)

    plsc.subcore_barrier()

    ones_c = ones_v.at[pl.ds(0, CK)]
    sems = (sm0, sm1, sm2, sm3)
    for b in range(4):
        pltpu.async_copy(ones_c, acc.at[idxm.at[b]], sems[b], add=True)

    def ring(k, carry):
        i = 4 * k
        for b in range(4):
            pltpu.make_async_copy(ones_c, acc.at[idxm.at[i + b]],
                                  sems[b]).wait()

            @pl.when(k + 1 < DCH // 4)
            def _():
                pltpu.async_copy(ones_c, acc.at[idxm.at[i + 4 + b]], sems[b],
                                 add=True)

        return carry

    lax.fori_loop(0, DCH // 4, ring, 0)
    plsc.subcore_barrier()

    pltpu.sync_copy(acc.at[pl.ds(base, ROWS_PER_TILE_PAD)], degv)

    def nbody(r, carry):
        dv = degv[pl.ds(r * 16, 16)]
        d = jnp.maximum(dv, 1.0)
        i32 = lax.bitcast_convert_type(d, jnp.int32)
        y = lax.bitcast_convert_type(jnp.int32(0x5F3759DF) - (i32 >> 1),
                                     jnp.float32)
        for _ in range(3):
            y = y * (1.5 - 0.5 * d * y * y)
        degv[pl.ds(r * 16, 16)] = jnp.where(dv > 0.0, y, 0.0)
        return carry

    lax.fori_loop(0, ROWS_PER_TILE_PAD // 16, nbody, 0)

    @pl.when(c == 0)
    def _():
        pltpu.sync_copy(degv, ns_hbm.at[pl.ds(base, ROWS_PER_TILE_PAD)])

    @pl.when(c != 0)
    def _():
        pltpu.sync_copy(degv, nd_hbm.at[pl.ds(base, ROWS_PER_TILE_PAD)])


@functools.lru_cache(maxsize=None)
def _deg_call():
    return pl.kernel(
        _deg_body,
        out_type=(jax.ShapeDtypeStruct((N_PAD,), jnp.float32),
                  jax.ShapeDtypeStruct((N_PAD,), jnp.float32)),
        mesh=_mesh(),
        scratch_types=[
            pltpu.VMEM_SHARED((N_PAD,), jnp.float32),
            pltpu.VMEM((DCH, CK), jnp.int32),
            pltpu.VMEM((112,), jnp.float32),
            pltpu.VMEM((ROWS_PER_TILE_PAD,), jnp.float32),
            pltpu.VMEM((ROWS_PER_TILE_PAD,), jnp.float32),
            pltpu.SemaphoreType.DMA,
            pltpu.SemaphoreType.DMA,
            pltpu.SemaphoreType.DMA,
            pltpu.SemaphoreType.DMA,
        ],
    )


def _gsh_body(FH, idx_halves, final, yh_hbm, src_hbm, dst_hbm, ns_hbm,
              nd_hbm, b_hbm, out_hbm, acc, ytab, idx_s, idx_d, rows0, rows1,
              rows2, rows3, ndv, bv, sg0, sg1, sg2, sg3, ss0, ss1, ss2, ss3):
    c = lax.axis_index("c")
    s = lax.axis_index("s")
    zero = jnp.zeros((16,), jnp.float32)

    def zb(r, carry):
        for j in range(FH // 16):
            rows0[r, pl.ds(j * 16, 16)] = zero
        return carry

    lax.fori_loop(0, CKP, zb, 0)
    base_rows = s * ROWS_PER_TILE_PAD
    for j in range(ROWS_PER_TILE_PAD // CKP):
        pltpu.sync_copy(rows0, acc.at[pl.ds(base_rows + j * CKP, CKP)])
    tslice = N_NODES // NS
    pltpu.sync_copy(yh_hbm.at[c, pl.ds(s * tslice, tslice)],
                    ytab.at[pl.ds(s * tslice, tslice)])
    nh = ECH // idx_halves

    rows = (rows0, rows1, rows2, rows3)
    sg = (sg0, sg1, sg2, sg3)
    ss = (ss0, ss1, ss2, ss3)

    def run_half(h):
        pltpu.sync_copy(src_hbm.at[s, pl.ds(h * nh, nh)], idx_s)
        pltpu.sync_copy(dst_hbm.at[s, pl.ds(h * nh, nh)], idx_d)
        if h == 0:
            plsc.subcore_barrier()
        for b in range(4):
            pltpu.async_copy(ytab.at[idx_s.at[b]], rows[b], sg[b])

        def quad(k, carry):
            i = 4 * k
            for b in range(4):
                m = i + b
                pltpu.make_async_copy(ytab.at[idx_s.at[m]], rows[b],
                                      sg[b]).wait()
                pltpu.async_copy(rows[b], acc.at[idx_d.at[m]], ss[b],
                                 add=True)
                b2 = (b + 2) % 4
                mm = m - 2

                @pl.when(jnp.logical_and(mm >= 0, mm + 4 < nh))
                def _():
                    pltpu.make_async_copy(rows[b2], acc.at[idx_d.at[mm]],
                                          ss[b2]).wait()
                    pltpu.async_copy(ytab.at[idx_s.at[mm + 4]], rows[b2],
                                     sg[b2])

            return carry

        lax.fori_loop(0, nh // 4, quad, 0)
        for b in range(4):
            pltpu.make_async_copy(rows[b], acc.at[idx_d.at[nh - 4 + b]],
                                  ss[b]).wait()

    for h in range(idx_halves):
        run_half(h)
    plsc.subcore_barrier()
    if not final:
        pltpu.sync_copy(acc.at[pl.ds(base_rows, ROWS_PER_TILE_PAD)],
                        out_hbm.at[c, pl.ds(base_rows, ROWS_PER_TILE_PAD)])
        return

    pltpu.sync_copy(b_hbm.at[pl.ds(c * FH, FH)], bv)
    for blk in range(ROWS_PER_TILE_PAD // CKP):
        rb = base_rows + blk * CKP
        pltpu.sync_copy(acc.at[pl.ds(rb, CKP)], rows0)
        pltpu.sync_copy(nd_hbm.at[pl.ds(rb, CKP)], ndv)

        def rowp(r16, carry):
            ndr16 = ndv[pl.ds(r16 * 16, 16)]
            for rr in range(16):
                r = r16 * 16 + rr
                nd_s = ndr16[rr]
                for j in range(FH // 16):
                    v = rows0[r, pl.ds(j * 16, 16)]
                    rows0[r, pl.ds(j * 16, 16)] = jnp.maximum(
                        v * nd_s + bv[pl.ds(j * 16, 16)], 0.0)
            return carry

        lax.fori_loop(0, CKP // 16, rowp, 0)
        rem = N_NODES % CKP

        @pl.when(rb + CKP <= N_NODES)
        def _():
            pltpu.sync_copy(rows0,
                            out_hbm.at[pl.ds(rb, CKP), pl.ds(c * FH, FH)])

        @pl.when(jnp.logical_and(rb < N_NODES, rb + CKP > N_NODES))
        def _():
            pltpu.sync_copy(rows0.at[pl.ds(0, rem)],
                            out_hbm.at[pl.ds(rb, rem), pl.ds(c * FH, FH)])


@functools.lru_cache(maxsize=None)
def _make_gsh(FH, idx_halves, final=False):
    if final:
        out_type = jax.ShapeDtypeStruct((N_NODES, 2 * FH), jnp.float32)
    else:
        out_type = jax.ShapeDtypeStruct((NC, N_PAD, FH), jnp.float32)
    return pl.kernel(
        functools.partial(_gsh_body, FH, idx_halves, final),
        out_type=out_type,
        mesh=_mesh(),
        scratch_types=[
            pltpu.VMEM_SHARED((N_PAD, FH), jnp.float32),
            pltpu.VMEM_SHARED((N_PAD, FH), jnp.float32),
            pltpu.VMEM((ECH // idx_halves, CKP), jnp.int32),
            pltpu.VMEM((ECH // idx_halves, CKP), jnp.int32),
            pltpu.VMEM((CKP, FH), jnp.float32),
            pltpu.VMEM((CKP, FH), jnp.float32),
            pltpu.VMEM((CKP, FH), jnp.float32),
            pltpu.VMEM((CKP, FH), jnp.float32),
            pltpu.VMEM((CKP,), jnp.float32),
            pltpu.VMEM((FH,), jnp.float32),
        ] + [pltpu.SemaphoreType.DMA] * 8,
        compiler_params=pltpu.CompilerParams(use_tc_tiling_on_sc=False),
    )


def _prep_idx(src, dst):
    e_tile = N_EDGES // NS
    padt = ECH * CKP - e_tile
    zpad = jnp.zeros((NS, padt), jnp.int32)
    srcp = jnp.concatenate([src.reshape(NS, e_tile), zpad],
                           axis=1).reshape(NS, ECH, CKP)
    dpad = jnp.full((NS, padt), N_NODES, jnp.int32)
    dstp = jnp.concatenate([dst.reshape(NS, e_tile), dpad],
                           axis=1).reshape(NS, ECH, CKP)
    return srcp, dstp


def _tc1_body(x_ref, ns_ref, w_ref, o_ref):
    z = jnp.dot(x_ref[...] * ns_ref[...], w_ref[...],
                preferred_element_type=jnp.float32)
    fh = z.shape[1] // 2
    o_ref[0] = z[:, :fh]
    o_ref[1] = z[:, fh:]


def _tc2_body(p_ref, nd_ref, b1_ref, ns_ref, w2_ref, o_ref):
    agg = jnp.concatenate([p_ref[0, :N_NODES], p_ref[1, :N_NODES]], axis=1)
    h = jnp.maximum(agg * nd_ref[...] + b1_ref[...], 0.0)
    z = jnp.dot(h * ns_ref[...], w2_ref[...],
                preferred_element_type=jnp.float32)
    fh = z.shape[1] // 2
    o_ref[0] = z[:, :fh]
    o_ref[1] = z[:, fh:]


def _tc1_call(x, ns, w1):
    return pl.pallas_call(
        _tc1_body,
        out_shape=jax.ShapeDtypeStruct((2, N_NODES, w1.shape[1] // 2),
                                       jnp.float32),
    )(x, ns, w1)


def _tc2_call(p, nd, b1, ns, w2):
    return pl.pallas_call(
        _tc2_body,
        out_shape=jax.ShapeDtypeStruct((2, N_NODES, w2.shape[1] // 2),
                                       jnp.float32),
    )(p, nd, b1, ns, w2)


def kernel(in_feat, edge_index, W1, b1, W2, b2):
    ei = edge_index.astype(jnp.int32)
    src = ei[0]
    dst = ei[1]
    srcd = src.reshape(NS, DCH, CK)
    dstd = dst.reshape(NS, DCH, CK)
    srcx, dstx = _prep_idx(src, dst)
    ns_pad, nd_pad = _deg_call()(srcd, dstd)
    ns = ns_pad[:N_NODES].reshape(N_NODES, 1)
    nd = nd_pad[:N_NODES].reshape(N_NODES, 1)
    y1h = _tc1_call(in_feat, ns, W1)
    p1 = _make_gsh(64, 4)(y1h, srcx, dstx, ns_pad, nd_pad, b1)
    y2h = _tc2_call(p1, nd, b1, ns, W2)
    return _make_gsh(32, 1, final=True)(y2h, srcx, dstx, ns_pad, nd_pad, b2)


# speedup vs baseline: 1.2290x; 1.0570x over previous
import functools

import jax
import jax.numpy as jnp
from jax import lax
from jax.experimental import pallas as pl
from jax.experimental.pallas import tpu as pltpu
from jax.experimental.pallas import tpu_sc as plsc

N_NODES = 10000
N_EDGES = 320000
NC = 2
NS = 16
N_PAD = 10240
ROWS_PER_TILE_PAD = N_PAD // NS
CK = 100
DCH = N_EDGES // NS // CK
CKP = 128
CKE = 125
ECH = 160


@functools.lru_cache(maxsize=None)
def _mesh():
    return plsc.VectorSubcoreMesh(core_axis_name="c", subcore_axis_name="s",
                                  num_cores=NC, num_subcores=NS)


def _deg_body(src_hbm, dst_hbm, ns_hbm, nd_hbm, acc, idxm, ones_v, degv, zv,
              sm0, sm1, sm2, sm3):
    c = lax.axis_index("c")
    s = lax.axis_index("s")
    one = jnp.ones((16,), jnp.float32)
    zero = jnp.zeros((16,), jnp.float32)
    for j in range(112 // 16):
        ones_v[pl.ds(j * 16, 16)] = one
    for j in range(ROWS_PER_TILE_PAD // 16):
        zv[pl.ds(j * 16, 16)] = zero
    base = s * ROWS_PER_TILE_PAD
    pltpu.sync_copy(zv, acc.at[pl.ds(base, ROWS_PER_TILE_PAD)])

    @pl.when(c == 0)
    def _():
        pltpu.sync_copy(src_hbm.at[s], idxm)

    @pl.when(c != 0)
    def _():
        pltpu.sync_copy(dst_hbm.at[s], idxm)

    plsc.subcore_barrier()

    ones_c = ones_v.at[pl.ds(0, CK)]
    sems = (sm0, sm1, sm2, sm3)
    for b in range(4):
        pltpu.async_copy(ones_c, acc.at[idxm.at[b]], sems[b], add=True)

    def ring(k, carry):
        i = 4 * k
        for b in range(4):
            pltpu.make_async_copy(ones_c, acc.at[idxm.at[i + b]],
                                  sems[b]).wait()

            @pl.when(k + 1 < DCH // 4)
            def _():
                pltpu.async_copy(ones_c, acc.at[idxm.at[i + 4 + b]], sems[b],
                                 add=True)

        return carry

    lax.fori_loop(0, DCH // 4, ring, 0)
    plsc.subcore_barrier()

    pltpu.sync_copy(acc.at[pl.ds(base, ROWS_PER_TILE_PAD)], degv)

    def nbody(r, carry):
        dv = degv[pl.ds(r * 16, 16)]
        d = jnp.maximum(dv, 1.0)
        i32 = lax.bitcast_convert_type(d, jnp.int32)
        y = lax.bitcast_convert_type(jnp.int32(0x5F3759DF) - (i32 >> 1),
                                     jnp.float32)
        for _ in range(3):
            y = y * (1.5 - 0.5 * d * y * y)
        degv[pl.ds(r * 16, 16)] = jnp.where(dv > 0.0, y, 0.0)
        return carry

    lax.fori_loop(0, ROWS_PER_TILE_PAD // 16, nbody, 0)

    @pl.when(c == 0)
    def _():
        pltpu.sync_copy(degv, ns_hbm.at[pl.ds(base, ROWS_PER_TILE_PAD)])

    @pl.when(c != 0)
    def _():
        pltpu.sync_copy(degv, nd_hbm.at[pl.ds(base, ROWS_PER_TILE_PAD)])


@functools.lru_cache(maxsize=None)
def _deg_call():
    return pl.kernel(
        _deg_body,
        out_type=(jax.ShapeDtypeStruct((N_PAD,), jnp.float32),
                  jax.ShapeDtypeStruct((N_PAD,), jnp.float32)),
        mesh=_mesh(),
        scratch_types=[
            pltpu.VMEM_SHARED((N_PAD,), jnp.float32),
            pltpu.VMEM((DCH, CK), jnp.int32),
            pltpu.VMEM((112,), jnp.float32),
            pltpu.VMEM((ROWS_PER_TILE_PAD,), jnp.float32),
            pltpu.VMEM((ROWS_PER_TILE_PAD,), jnp.float32),
            pltpu.SemaphoreType.DMA,
            pltpu.SemaphoreType.DMA,
            pltpu.SemaphoreType.DMA,
            pltpu.SemaphoreType.DMA,
        ],
    )


def _gsh_body(FH, idx_halves, final, yh_hbm, src_hbm, dst_hbm, ns_hbm,
              nd_hbm, b_hbm, out_hbm, acc, ytab, idx_s, idx_d, rows0, rows1,
              rows2, rows3, ndv, bv, sg0, sg1, sg2, sg3, ss0, ss1, ss2, ss3):
    c = lax.axis_index("c")
    s = lax.axis_index("s")
    zero = jnp.zeros((16,), jnp.float32)

    def zb(r, carry):
        for j in range(FH // 16):
            rows0[r, pl.ds(j * 16, 16)] = zero
        return carry

    lax.fori_loop(0, CKP, zb, 0)
    base_rows = s * ROWS_PER_TILE_PAD
    for j in range(ROWS_PER_TILE_PAD // CKP):
        pltpu.sync_copy(rows0, acc.at[pl.ds(base_rows + j * CKP, CKP)])
    tslice = N_NODES // NS
    pltpu.sync_copy(yh_hbm.at[c, pl.ds(s * tslice, tslice)],
                    ytab.at[pl.ds(s * tslice, tslice)])
    nh = ECH // idx_halves

    rows = (rows0, rows1, rows2, rows3)
    sg = (sg0, sg1, sg2, sg3)
    ss = (ss0, ss1, ss2, ss3)

    def run_half(h):
        pltpu.sync_copy(src_hbm.at[s, pl.ds(h * nh, nh)], idx_s)
        pltpu.sync_copy(dst_hbm.at[s, pl.ds(h * nh, nh)], idx_d)
        if h == 0:
            plsc.subcore_barrier()
        for b in range(4):
            pltpu.async_copy(ytab.at[idx_s.at[b]],
                             rows[b].at[pl.ds(0, CKE)], sg[b])

        def quad(k, carry):
            i = 4 * k
            for b in range(4):
                m = i + b
                pltpu.make_async_copy(ytab.at[idx_s.at[m]],
                                      rows[b].at[pl.ds(0, CKE)],
                                      sg[b]).wait()
                pltpu.async_copy(rows[b].at[pl.ds(0, CKE)],
                                 acc.at[idx_d.at[m]], ss[b], add=True)
                b2 = (b + 2) % 4
                mm = m - 2

                @pl.when(jnp.logical_and(mm >= 0, mm + 4 < nh))
                def _():
                    pltpu.make_async_copy(rows[b2].at[pl.ds(0, CKE)],
                                          acc.at[idx_d.at[mm]],
                                          ss[b2]).wait()
                    pltpu.async_copy(ytab.at[idx_s.at[mm + 4]],
                                     rows[b2].at[pl.ds(0, CKE)], sg[b2])

            return carry

        lax.fori_loop(0, nh // 4, quad, 0)
        for b in range(4):
            pltpu.make_async_copy(rows[b].at[pl.ds(0, CKE)],
                                  acc.at[idx_d.at[nh - 4 + b]], ss[b]).wait()

    for h in range(idx_halves):
        run_half(h)
    plsc.subcore_barrier()
    if not final:
        pltpu.sync_copy(acc.at[pl.ds(base_rows, ROWS_PER_TILE_PAD)],
                        out_hbm.at[c, pl.ds(base_rows, ROWS_PER_TILE_PAD)])
        return

    pltpu.sync_copy(b_hbm.at[pl.ds(c * FH, FH)], bv)
    for blk in range(ROWS_PER_TILE_PAD // CKP):
        rb = base_rows + blk * CKP
        pltpu.sync_copy(acc.at[pl.ds(rb, CKP)], rows0)
        pltpu.sync_copy(nd_hbm.at[pl.ds(rb, CKP)], ndv)

        def rowp(r16, carry):
            ndr16 = ndv[pl.ds(r16 * 16, 16)]
            for rr in range(16):
                r = r16 * 16 + rr
                nd_s = ndr16[rr]
                for j in range(FH // 16):
                    v = rows0[r, pl.ds(j * 16, 16)]
                    rows0[r, pl.ds(j * 16, 16)] = jnp.maximum(
                        v * nd_s + bv[pl.ds(j * 16, 16)], 0.0)
            return carry

        lax.fori_loop(0, CKP // 16, rowp, 0)
        rem = N_NODES % CKP

        @pl.when(rb + CKP <= N_NODES)
        def _():
            pltpu.sync_copy(rows0,
                            out_hbm.at[pl.ds(rb, CKP), pl.ds(c * FH, FH)])

        @pl.when(jnp.logical_and(rb < N_NODES, rb + CKP > N_NODES))
        def _():
            pltpu.sync_copy(rows0.at[pl.ds(0, rem)],
                            out_hbm.at[pl.ds(rb, rem), pl.ds(c * FH, FH)])


@functools.lru_cache(maxsize=None)
def _make_gsh(FH, idx_halves, final=False):
    if final:
        out_type = jax.ShapeDtypeStruct((N_NODES, 2 * FH), jnp.float32)
    else:
        out_type = jax.ShapeDtypeStruct((NC, N_PAD, FH), jnp.float32)
    return pl.kernel(
        functools.partial(_gsh_body, FH, idx_halves, final),
        out_type=out_type,
        mesh=_mesh(),
        scratch_types=[
            pltpu.VMEM_SHARED((N_PAD, FH), jnp.float32),
            pltpu.VMEM_SHARED((N_PAD, FH), jnp.float32),
            pltpu.VMEM((ECH // idx_halves, CKE), jnp.int32),
            pltpu.VMEM((ECH // idx_halves, CKE), jnp.int32),
            pltpu.VMEM((CKP, FH), jnp.float32),
            pltpu.VMEM((CKP, FH), jnp.float32),
            pltpu.VMEM((CKP, FH), jnp.float32),
            pltpu.VMEM((CKP, FH), jnp.float32),
            pltpu.VMEM((CKP,), jnp.float32),
            pltpu.VMEM((FH,), jnp.float32),
        ] + [pltpu.SemaphoreType.DMA] * 8,
        compiler_params=pltpu.CompilerParams(use_tc_tiling_on_sc=False),
    )


def _prep_idx(src, dst):
    return src.reshape(NS, ECH, CKE), dst.reshape(NS, ECH, CKE)


def _tc1_body(x_ref, ns_ref, w_ref, o_ref):
    z = jnp.dot(x_ref[...] * ns_ref[...], w_ref[...],
                preferred_element_type=jnp.float32)
    fh = z.shape[1] // 2
    o_ref[0] = z[:, :fh]
    o_ref[1] = z[:, fh:]


def _tc2_body(p_ref, nd_ref, b1_ref, ns_ref, w2_ref, o_ref):
    agg = jnp.concatenate([p_ref[0, :N_NODES], p_ref[1, :N_NODES]], axis=1)
    h = jnp.maximum(agg * nd_ref[...] + b1_ref[...], 0.0)
    z = jnp.dot(h * ns_ref[...], w2_ref[...],
                preferred_element_type=jnp.float32)
    fh = z.shape[1] // 2
    o_ref[0] = z[:, :fh]
    o_ref[1] = z[:, fh:]


def _tc1_call(x, ns, w1):
    return pl.pallas_call(
        _tc1_body,
        out_shape=jax.ShapeDtypeStruct((2, N_NODES, w1.shape[1] // 2),
                                       jnp.float32),
    )(x, ns, w1)


def _tc2_call(p, nd, b1, ns, w2):
    return pl.pallas_call(
        _tc2_body,
        out_shape=jax.ShapeDtypeStruct((2, N_NODES, w2.shape[1] // 2),
                                       jnp.float32),
    )(p, nd, b1, ns, w2)


def kernel(in_feat, edge_index, W1, b1, W2, b2):
    ei = edge_index.astype(jnp.int32)
    src = ei[0]
    dst = ei[1]
    srcd = src.reshape(NS, DCH, CK)
    dstd = dst.reshape(NS, DCH, CK)
    srcx, dstx = _prep_idx(src, dst)
    ns_pad, nd_pad = _deg_call()(srcd, dstd)
    ns = ns_pad[:N_NODES].reshape(N_NODES, 1)
    nd = nd_pad[:N_NODES].reshape(N_NODES, 1)
    y1h = _tc1_call(in_feat, ns, W1)
    p1 = _make_gsh(64, 4)(y1h, srcx, dstx, ns_pad, nd_pad, b1)
    y2h = _tc2_call(p1, nd, b1, ns, W2)
    return _make_gsh(32, 1, final=True)(y2h, srcx, dstx, ns_pad, nd_pad, b2)
